# baseline (speedup 1.0000x reference)
import functools

import jax
import jax.numpy as jnp
from jax import lax
from jax.experimental import pallas as pl
from jax.experimental.pallas import tpu as pltpu
from jax.experimental.pallas import tpu_sc as plsc

D = 64
NC, NS = 2, 16
NW = NC * NS
PB = 2
XB = 8
LANES = 16


@functools.lru_cache(maxsize=None)
def _make_kernel(B, L, V):
    lanes_per_w = B // NW
    n_blocks = L // XB
    n_sub = XB // PB
    mesh = plsc.VectorSubcoreMesh(core_axis_name="c", subcore_axis_name="s")

    @functools.partial(
        pl.kernel,
        mesh=mesh,
        out_type=jax.ShapeDtypeStruct((L, D, B), jnp.float32),
        scratch_types=[
            pltpu.VMEM((XB, lanes_per_w), jnp.int32),
            pltpu.VMEM((PB, lanes_per_w), jnp.int32),
            pltpu.VMEM((PB * lanes_per_w, 128), jnp.float32),
            pltpu.VMEM((PB, D, lanes_per_w), jnp.float32),
            pltpu.VMEM((L, 128), jnp.float32),
            pltpu.SemaphoreType.DMA,
        ],
        compiler_params=pltpu.CompilerParams(
            use_tc_tiling_on_sc=True, needs_layout_passes=False
        ),
    )
    def k(xt_hbm, t2_hbm, pe_hbm, out_hbm, xb_v, idx_v, rows_v, ob_v, pe_v, sem):
        wid = lax.axis_index("s") * NC + lax.axis_index("c")
        lane0 = wid * lanes_per_w
        iota = lax.iota(jnp.int32, LANES)
        pltpu.sync_copy(pe_hbm, pe_v)

        def block_body(lb, carry):
            pltpu.sync_copy(
                xt_hbm.at[pl.ds(lb * XB, XB), pl.ds(lane0, lanes_per_w)], xb_v
            )

            def chunk_body(kk, carry2):
                l0 = lb * XB + kk * PB
                for l in range(PB):
                    for g in range(lanes_per_w // LANES):
                        sl = pl.ds(LANES * g, LANES)
                        idx_v[l, sl] = xb_v[kk * PB + l, sl] >> 1
                cps = [
                    pltpu.async_copy(
                        t2_hbm.at[idx_v.at[l]],
                        rows_v.at[pl.ds(l * lanes_per_w, lanes_per_w), :],
                        sem,
                    )
                    for l in range(PB)
                ]
                for cp in cps:
                    cp.wait()

                for l in range(PB):
                    brs, pcs = [], []
                    for g in range(lanes_per_w // LANES):
                        xv = xb_v[kk * PB + l, pl.ds(LANES * g, LANES)]
                        pcs.append((xv & 1) << 6)
                        brs.append(l * lanes_per_w + LANES * g + iota)

                    def col_body(c, carry3):
                        pv = plsc.load_gather(
                            pe_v, [jnp.full((LANES,), l0 + l, jnp.int32),
                                   jnp.full((LANES,), c, jnp.int32)]
                        )
                        for g in range(lanes_per_w // LANES):
                            val = plsc.load_gather(
                                rows_v, [brs[g], pcs[g] + c]
                            )
                            ob_v[l, c, pl.ds(LANES * g, LANES)] = val + pv
                        return carry3

                    lax.fori_loop(0, D, col_body, 0)

                pltpu.sync_copy(
                    ob_v,
                    out_hbm.at[pl.ds(l0, PB), :, pl.ds(lane0, lanes_per_w)],
                )
                return carry2

            lax.fori_loop(0, n_sub, chunk_body, 0)
            return carry

        lax.fori_loop(0, n_blocks, block_body, 0)

    return k


def kernel(x, table, pe):
    B, L = x.shape
    V = table.shape[0]
    xt = x.astype(jnp.int32).T
    t2 = table.reshape(V // 2, 128)
    pe128 = jnp.pad(pe[0, :L, :], ((0, 0), (0, 128 - D)))
    out_t = _make_kernel(B, L, V)(xt, t2, pe128)
    return out_t.transpose(2, 0, 1)
